# baseline (speedup 1.0000x reference)
import functools

import jax
import jax.numpy as jnp
from jax import lax
from jax.experimental import pallas as pl
from jax.experimental.pallas import tpu as pltpu, tpu_sc as plsc

_BINS = 10
_NC, _NS, _L = 2, 16, 16
_NW = _NC * _NS
_TOTAL = 4194304 * 2
_PER_W = _TOTAL // _NW
_CHUNK = 32768
_NCH = _PER_W // _CHUNK
_VREGS = _CHUNK // _L

_P5 = (0.03010262501166993, -0.130119415391255, 0.2833043245174043,
       -0.4891568472023034, 0.9990104466294587, 2.2117031200142952e-05)


def _ghm_body(x_hbm, t_hbm, out_hbm, xb0, xb1, tb0, tb1, cnt, sm,
              semx0, semx1, semt0, semt1):
    c = lax.axis_index("c")
    s = lax.axis_index("s")
    wid = s * _NC + c
    base = wid * _PER_W
    sbase = wid * (_PER_W // 2)

    xbuf = (xb0, xb1)
    tbuf = (tb0, tb1)
    semx = (semx0, semx1)
    semt = (semt0, semt1)

    zero16 = jnp.zeros((_L,), jnp.float32)
    for r in range(_BINS):
        cnt[r] = zero16
        sm[r] = zero16

    iota = lax.iota(jnp.int32, _L)
    half = lax.shift_right_logical(iota, 1)
    jvec = lax.bitwise_and(iota, 1)
    ones = jnp.ones((_L,), jnp.float32)
    zeros = zero16

    def start(g):
        b = g % 2
        dx = pltpu.async_copy(
            x_hbm.at[pl.ds(base + g * _CHUNK, _CHUNK)], xbuf[b], semx[b])
        dt = pltpu.async_copy(
            t_hbm.at[pl.ds(sbase + g * (_CHUNK // 2), _CHUNK // 2)],
            tbuf[b], semt[b])
        return dx, dt

    pending = {0: start(0)}
    for g in range(_NCH):
        b = g % 2
        if g + 1 < _NCH:
            pending[g + 1] = start(g + 1)
        dx, dt = pending.pop(g)
        dx.wait()
        dt.wait()

        def inner(i, _, xr=xbuf[b], tr=tbuf[b]):
            v = xr[pl.ds(i * _L, _L)]
            tg = plsc.load_gather(tr, [i * (_L // 2) + half])
            xt = jnp.where(tg == jvec, -v, v)
            e = jnp.exp(-jnp.abs(xt))
            num = jnp.where(xt >= 0.0, ones, e)
            g_ = num / (1.0 + e)
            bin_ = (g_ * jnp.float32(_BINS - 0.0001)).astype(jnp.int32)
            p = jnp.full((_L,), _P5[0], jnp.float32)
            for coef in _P5[1:]:
                p = p * e + jnp.float32(coef)
            bce = jnp.maximum(xt, 0.0) + p
            plsc.addupdate_scatter(cnt, [bin_, iota], ones)
            plsc.addupdate_scatter(sm, [bin_, iota], bce)
            return 0

        lax.fori_loop(0, _VREGS, inner, 0)

    pltpu.sync_copy(cnt, out_hbm.at[wid, 0])
    pltpu.sync_copy(sm, out_hbm.at[wid, 1])


@functools.partial(
    pl.kernel,
    out_type=jax.ShapeDtypeStruct((_NW, 2, _BINS, _L), jnp.float32),
    mesh=plsc.VectorSubcoreMesh(
        core_axis_name="c", subcore_axis_name="s",
        num_cores=_NC, num_subcores=_NS),
    compiler_params=pltpu.CompilerParams(needs_layout_passes=False),
    scratch_types=[
        pltpu.VMEM((_CHUNK,), jnp.float32),
        pltpu.VMEM((_CHUNK,), jnp.float32),
        pltpu.VMEM((_CHUNK // 2,), jnp.int32),
        pltpu.VMEM((_CHUNK // 2,), jnp.int32),
        pltpu.VMEM((_BINS, _L), jnp.float32),
        pltpu.VMEM((_BINS, _L), jnp.float32),
        pltpu.SemaphoreType.DMA,
        pltpu.SemaphoreType.DMA,
        pltpu.SemaphoreType.DMA,
        pltpu.SemaphoreType.DMA,
    ],
)
def _ghm_pass(x_hbm, t_hbm, out_hbm, *rest):
    _ghm_body(x_hbm, t_hbm, out_hbm, *rest)


def kernel(x, target):
    x_flat = x.reshape(-1)
    tgt = target.astype(jnp.int32)
    parts = _ghm_pass(x_flat, tgt)
    cnt = parts[:, 0].sum(axis=(0, 2))
    sums = parts[:, 1].sum(axis=(0, 2))
    nonempty = jnp.sum(cnt > 0).astype(jnp.float32)
    gd = jnp.clip(cnt * nonempty, 0.0001, None)
    return jnp.sum(sums / gd)


# speedup vs baseline: 1.0727x; 1.0727x over previous
import functools

import jax
import jax.numpy as jnp
from jax import lax
from jax.experimental import pallas as pl
from jax.experimental.pallas import tpu as pltpu, tpu_sc as plsc

_BINS = 10
_NC, _NS, _L = 2, 16, 16
_NW = _NC * _NS
_TOTAL = 4194304 * 2
_PER_W = _TOTAL // _NW
_CHUNK = 32768
_NCH = _PER_W // _CHUNK
_VREGS = _CHUNK // _L

_P4 = (-0.05486285286206755, 0.21641043832783038, -0.46407258044713595,
       0.995427338257992, 0.00014151217537860064)


def _ghm_body(x_hbm, t_hbm, out_hbm, xb0, xb1, tb0, tb1, cnt, sm,
              semx0, semx1, semt0, semt1):
    c = lax.axis_index("c")
    s = lax.axis_index("s")
    wid = s * _NC + c
    base = wid * _PER_W
    sbase = wid * (_PER_W // 2)

    xbuf = (xb0, xb1)
    tbuf = (tb0, tb1)
    semx = (semx0, semx1)
    semt = (semt0, semt1)

    zero16 = jnp.zeros((_L,), jnp.float32)
    for r in range(_BINS):
        cnt[r] = zero16
        sm[r] = zero16

    iota = lax.iota(jnp.int32, _L)
    half = lax.shift_right_logical(iota, 1)
    jvec = lax.bitwise_and(iota, 1)
    ones = jnp.ones((_L,), jnp.float32)
    zeros = zero16

    def start(g):
        b = g % 2
        dx = pltpu.async_copy(
            x_hbm.at[pl.ds(base + g * _CHUNK, _CHUNK)], xbuf[b], semx[b])
        dt = pltpu.async_copy(
            t_hbm.at[pl.ds(sbase + g * (_CHUNK // 2), _CHUNK // 2)],
            tbuf[b], semt[b])
        return dx, dt

    pending = {0: start(0)}
    for g in range(_NCH):
        b = g % 2
        if g + 1 < _NCH:
            pending[g + 1] = start(g + 1)
        dx, dt = pending.pop(g)
        dx.wait()
        dt.wait()

        @plsc.parallel_loop(0, _VREGS, unroll=8)
        def inner(i, xr=xbuf[b], tr=tbuf[b]):
            v = xr[pl.ds(i * _L, _L)]
            tg = plsc.load_gather(tr, [i * (_L // 2) + half])
            xt = jnp.where(tg == jvec, -v, v)
            e = jnp.exp(-jnp.abs(xt))
            r = 1.0 / (1.0 + e)
            g_ = jnp.where(xt >= 0.0, r, 1.0 - r)
            bin_ = (g_ * jnp.float32(_BINS - 0.0001)).astype(jnp.int32)
            p = jnp.full((_L,), _P4[0], jnp.float32)
            for coef in _P4[1:]:
                p = p * e + jnp.float32(coef)
            bce = jnp.maximum(xt, 0.0) + p
            plsc.addupdate_scatter(cnt, [bin_, iota], ones)
            plsc.addupdate_scatter(sm, [bin_, iota], bce)

    pltpu.sync_copy(cnt, out_hbm.at[wid, 0])
    pltpu.sync_copy(sm, out_hbm.at[wid, 1])


@functools.partial(
    pl.kernel,
    out_type=jax.ShapeDtypeStruct((_NW, 2, _BINS, _L), jnp.float32),
    mesh=plsc.VectorSubcoreMesh(
        core_axis_name="c", subcore_axis_name="s",
        num_cores=_NC, num_subcores=_NS),
    compiler_params=pltpu.CompilerParams(needs_layout_passes=False),
    scratch_types=[
        pltpu.VMEM((_CHUNK,), jnp.float32),
        pltpu.VMEM((_CHUNK,), jnp.float32),
        pltpu.VMEM((_CHUNK // 2,), jnp.int32),
        pltpu.VMEM((_CHUNK // 2,), jnp.int32),
        pltpu.VMEM((_BINS, _L), jnp.float32),
        pltpu.VMEM((_BINS, _L), jnp.float32),
        pltpu.SemaphoreType.DMA,
        pltpu.SemaphoreType.DMA,
        pltpu.SemaphoreType.DMA,
        pltpu.SemaphoreType.DMA,
    ],
)
def _ghm_pass(x_hbm, t_hbm, out_hbm, *rest):
    _ghm_body(x_hbm, t_hbm, out_hbm, *rest)


def kernel(x, target):
    x_flat = x.reshape(-1)
    tgt = target.astype(jnp.int32)
    parts = _ghm_pass(x_flat, tgt)
    cnt = parts[:, 0].sum(axis=(0, 2))
    sums = parts[:, 1].sum(axis=(0, 2))
    nonempty = jnp.sum(cnt > 0).astype(jnp.float32)
    gd = jnp.clip(cnt * nonempty, 0.0001, None)
    return jnp.sum(sums / gd)


# speedup vs baseline: 50.1306x; 46.7311x over previous
import functools

import jax
import jax.numpy as jnp
from jax import lax
from jax.experimental import pallas as pl
from jax.experimental.pallas import tpu as pltpu, tpu_sc as plsc

_BINS = 10
_NC, _NS, _L = 2, 16, 16
_NW = _NC * _NS
_NSAMP = 4194304
_TOTAL = _NSAMP * 2
_PER_W = _TOTAL // _NW
_CHUNK = 32768
_NCH = _PER_W // _CHUNK
_PAIRS = _CHUNK // (2 * _L)

_P4 = (-0.05486285286206755, 0.21641043832783038, -0.46407258044713595,
       0.995427338257992, 0.00014151217537860064)


def _ghm_body(x_hbm, t_hbm, out_hbm, xb0, xb1, tb0, tb1, cnt, sm,
              semx0, semx1, semt0, semt1):
    c = lax.axis_index("c")
    s = lax.axis_index("s")
    wid = s * _NC + c
    base = wid * _PER_W
    sbase = wid * (_PER_W // 2)

    xbuf = (xb0, xb1)
    tbuf = (tb0, tb1)
    semx = (semx0, semx1)
    semt = (semt0, semt1)

    zero16 = jnp.zeros((_L,), jnp.float32)
    for r in range(_BINS):
        cnt[r] = zero16
        sm[r] = zero16

    iota = lax.iota(jnp.int32, _L)
    ones = jnp.ones((_L,), jnp.float32)

    def start(g):
        b = g % 2
        dx = pltpu.async_copy(
            x_hbm.at[pl.ds(base + g * _CHUNK, _CHUNK)], xbuf[b], semx[b])
        dt = pltpu.async_copy(
            t_hbm.at[pl.ds(sbase + g * (_CHUNK // 2), _CHUNK // 2)],
            tbuf[b], semt[b])
        return dx, dt

    def accum(xt):
        e = jnp.exp(-jnp.abs(xt))
        r = 1.0 / (1.0 + e)
        g_ = jnp.where(xt >= 0.0, r, 1.0 - r)
        bin_ = (g_ * jnp.float32(_BINS - 0.0001)).astype(jnp.int32)
        p = jnp.full((_L,), _P4[0], jnp.float32)
        for coef in _P4[1:]:
            p = p * e + jnp.float32(coef)
        bce = jnp.maximum(xt, 0.0) + p
        plsc.addupdate_scatter(cnt, [bin_, iota], ones)
        plsc.addupdate_scatter(sm, [bin_, iota], bce)

    pending = {0: start(0)}
    for g in range(_NCH):
        b = g % 2
        if g + 1 < _NCH:
            pending[g + 1] = start(g + 1)
        dx, dt = pending.pop(g)
        dx.wait()
        dt.wait()

        @plsc.parallel_loop(0, _PAIRS, unroll=4)
        def inner(i, xr=xbuf[b], tr=tbuf[b]):
            blk = lax.shift_right_logical(i, 3)
            sub = lax.bitwise_and(i, 7)
            off0 = blk * 256 + sub * _L
            x0 = xr[pl.ds(off0, _L)]
            x1 = xr[pl.ds(off0 + 128, _L)]
            tg = tr[pl.ds(i * _L, _L)]
            y0 = tg == 0
            accum(jnp.where(y0, -x0, x0))
            accum(jnp.where(y0, x1, -x1))

    pltpu.sync_copy(cnt, out_hbm.at[wid, 0])
    pltpu.sync_copy(sm, out_hbm.at[wid, 1])


@functools.partial(
    pl.kernel,
    out_type=jax.ShapeDtypeStruct((_NW, 2, _BINS, _L), jnp.float32),
    mesh=plsc.VectorSubcoreMesh(
        core_axis_name="c", subcore_axis_name="s",
        num_cores=_NC, num_subcores=_NS),
    compiler_params=pltpu.CompilerParams(needs_layout_passes=False),
    scratch_types=[
        pltpu.VMEM((_CHUNK,), jnp.float32),
        pltpu.VMEM((_CHUNK,), jnp.float32),
        pltpu.VMEM((_CHUNK // 2,), jnp.int32),
        pltpu.VMEM((_CHUNK // 2,), jnp.int32),
        pltpu.VMEM((_BINS, _L), jnp.float32),
        pltpu.VMEM((_BINS, _L), jnp.float32),
        pltpu.SemaphoreType.DMA,
        pltpu.SemaphoreType.DMA,
        pltpu.SemaphoreType.DMA,
        pltpu.SemaphoreType.DMA,
    ],
)
def _ghm_pass(x_hbm, t_hbm, out_hbm, *rest):
    _ghm_body(x_hbm, t_hbm, out_hbm, *rest)


def kernel(x, target):
    x_lin = x.reshape(32768, 128, 2).transpose(0, 2, 1).reshape(-1)
    tgt = target.astype(jnp.int32)
    parts = _ghm_pass(x_lin, tgt)
    cnt = parts[:, 0].sum(axis=(0, 2))
    sums = parts[:, 1].sum(axis=(0, 2))
    nonempty = jnp.sum(cnt > 0).astype(jnp.float32)
    gd = jnp.clip(cnt * nonempty, 0.0001, None)
    return jnp.sum(sums / gd)
